# initial kernel scaffold (unmeasured)
import jax
import jax.numpy as jnp
from jax import lax
from jax.experimental import pallas as pl
from jax.experimental.pallas import tpu as pltpu


def kernel(
    x,
):
    def body(*refs):
        pass

    out_shape = jax.ShapeDtypeStruct(..., jnp.float32)
    return pl.pallas_call(body, out_shape=out_shape)(...)


# baseline (device time: 19680 ns/iter reference)
import jax
import jax.numpy as jnp
from jax import lax
from jax.experimental import pallas as pl
from jax.experimental.pallas import tpu as pltpu


def kernel(x):
    m, n = x.shape

    def body(x_ref, out_ref, comm_ref, send_sems, recv_sems):
        my_x = lax.axis_index("x")
        my_y = lax.axis_index("y")
        x_nbr = (1 - my_x, my_y)
        y_nbr = (my_x, 1 - my_y)

        barrier_sem = pltpu.get_barrier_semaphore()
        for nbr in (x_nbr, y_nbr):
            pl.semaphore_signal(
                barrier_sem, inc=1,
                device_id=nbr, device_id_type=pl.DeviceIdType.MESH,
            )
        pl.semaphore_wait(barrier_sem, 2)

        comm_ref[0, :, :] = x_ref[:, :].astype(jnp.bfloat16)

        rdma1 = pltpu.make_async_remote_copy(
            src_ref=comm_ref.at[0],
            dst_ref=comm_ref.at[1],
            send_sem=send_sems.at[0],
            recv_sem=recv_sems.at[0],
            device_id=x_nbr,
            device_id_type=pl.DeviceIdType.MESH,
        )
        rdma1.start()
        rdma1.wait()

        red = comm_ref[0, :, :] + comm_ref[1, :, :]
        comm_ref[2, :, :] = red
        out_ref[:, pl.ds(my_y * n, n)] = red

        rdma2 = pltpu.make_async_remote_copy(
            src_ref=comm_ref.at[2],
            dst_ref=comm_ref.at[3],
            send_sem=send_sems.at[1],
            recv_sem=recv_sems.at[1],
            device_id=y_nbr,
            device_id_type=pl.DeviceIdType.MESH,
        )
        rdma2.start()
        rdma2.wait()
        out_ref[:, pl.ds((1 - my_y) * n, n)] = comm_ref[3, :, :]

    return pl.pallas_call(
        body,
        out_shape=jax.ShapeDtypeStruct((m, 2 * n), jnp.bfloat16),
        in_specs=[pl.BlockSpec(memory_space=pltpu.VMEM)],
        out_specs=pl.BlockSpec(memory_space=pltpu.VMEM),
        scratch_shapes=[
            pltpu.VMEM((4, m, n), jnp.bfloat16),
            pltpu.SemaphoreType.DMA((2,)),
            pltpu.SemaphoreType.DMA((2,)),
        ],
        compiler_params=pltpu.CompilerParams(collective_id=0),
    )(x)


# device time: 15490 ns/iter; 1.2705x vs baseline; 1.2705x over previous
import jax
import jax.numpy as jnp
from jax import lax
from jax.experimental import pallas as pl
from jax.experimental.pallas import tpu as pltpu

C = 4


def kernel(x):
    m, n = x.shape
    mc = m // C

    def body(x_ref, out_ref, comm_ref, s1, r1, s2, r2):
        my_x = lax.axis_index("x")
        my_y = lax.axis_index("y")
        x_nbr = (1 - my_x, my_y)
        y_nbr = (my_x, 1 - my_y)

        barrier_sem = pltpu.get_barrier_semaphore()
        for nbr in (x_nbr, y_nbr):
            pl.semaphore_signal(
                barrier_sem, inc=1,
                device_id=nbr, device_id_type=pl.DeviceIdType.MESH,
            )
        pl.semaphore_wait(barrier_sem, 2)

        def hop1(c):
            return pltpu.make_async_remote_copy(
                src_ref=comm_ref.at[0, c],
                dst_ref=comm_ref.at[1, c],
                send_sem=s1.at[c],
                recv_sem=r1.at[c],
                device_id=x_nbr,
                device_id_type=pl.DeviceIdType.MESH,
            )

        def hop2(c):
            return pltpu.make_async_remote_copy(
                src_ref=comm_ref.at[2, c],
                dst_ref=comm_ref.at[3, c],
                send_sem=s2.at[c],
                recv_sem=r2.at[c],
                device_id=y_nbr,
                device_id_type=pl.DeviceIdType.MESH,
            )

        for c in range(C):
            comm_ref[0, c, :, :] = x_ref[pl.ds(c * mc, mc), :].astype(
                jnp.bfloat16
            )
            hop1(c).start()

        for c in range(C):
            hop1(c).wait_recv()
            red = comm_ref[0, c, :, :] + comm_ref[1, c, :, :]
            comm_ref[2, c, :, :] = red
            out_ref[pl.ds(c * mc, mc), pl.ds(my_y * n, n)] = red
            hop2(c).start()

        for c in range(C):
            hop2(c).wait_recv()
            out_ref[pl.ds(c * mc, mc), pl.ds((1 - my_y) * n, n)] = (
                comm_ref[3, c, :, :]
            )
        for c in range(C):
            hop1(c).wait_send()
            hop2(c).wait_send()

    return pl.pallas_call(
        body,
        out_shape=jax.ShapeDtypeStruct((m, 2 * n), jnp.bfloat16),
        in_specs=[pl.BlockSpec(memory_space=pltpu.VMEM)],
        out_specs=pl.BlockSpec(memory_space=pltpu.VMEM),
        scratch_shapes=[
            pltpu.VMEM((4, C, mc, n), jnp.bfloat16),
            pltpu.SemaphoreType.DMA((C,)),
            pltpu.SemaphoreType.DMA((C,)),
            pltpu.SemaphoreType.DMA((C,)),
            pltpu.SemaphoreType.DMA((C,)),
        ],
        compiler_params=pltpu.CompilerParams(collective_id=0),
    )(x)


# device time: 14907 ns/iter; 1.3202x vs baseline; 1.0391x over previous
import jax
import jax.numpy as jnp
from jax import lax
from jax.experimental import pallas as pl
from jax.experimental.pallas import tpu as pltpu

C = 8


def kernel(x):
    m, n = x.shape
    mc = m // C

    def body(x_ref, out_ref, comm_ref, s1, r1, s2, r2):
        my_x = lax.axis_index("x")
        my_y = lax.axis_index("y")
        x_nbr = (1 - my_x, my_y)
        y_nbr = (my_x, 1 - my_y)

        barrier_sem = pltpu.get_barrier_semaphore()
        for nbr in (x_nbr, y_nbr):
            pl.semaphore_signal(
                barrier_sem, inc=1,
                device_id=nbr, device_id_type=pl.DeviceIdType.MESH,
            )
        pl.semaphore_wait(barrier_sem, 2)

        def hop1(c):
            return pltpu.make_async_remote_copy(
                src_ref=comm_ref.at[0, c],
                dst_ref=comm_ref.at[1, c],
                send_sem=s1.at[c],
                recv_sem=r1.at[c],
                device_id=x_nbr,
                device_id_type=pl.DeviceIdType.MESH,
            )

        def hop2(c):
            blk = out_ref.at[pl.ds(c * mc, mc), pl.ds(my_y * n, n)]
            return pltpu.make_async_remote_copy(
                src_ref=blk,
                dst_ref=blk,
                send_sem=s2.at[c],
                recv_sem=r2.at[c],
                device_id=y_nbr,
                device_id_type=pl.DeviceIdType.MESH,
            )

        for c in range(C):
            comm_ref[0, c, :, :] = x_ref[pl.ds(c * mc, mc), :].astype(
                jnp.bfloat16
            )
            hop1(c).start()

        for c in range(C):
            hop1(c).wait_recv()
            out_ref[pl.ds(c * mc, mc), pl.ds(my_y * n, n)] = (
                comm_ref[0, c, :, :] + comm_ref[1, c, :, :]
            )
            hop2(c).start()

        for c in range(C):
            hop2(c).wait_recv()
        for c in range(C):
            hop1(c).wait_send()
            hop2(c).wait_send()

    return pl.pallas_call(
        body,
        out_shape=jax.ShapeDtypeStruct((m, 2 * n), jnp.bfloat16),
        in_specs=[pl.BlockSpec(memory_space=pltpu.VMEM)],
        out_specs=pl.BlockSpec(memory_space=pltpu.VMEM),
        scratch_shapes=[
            pltpu.VMEM((2, C, mc, n), jnp.bfloat16),
            pltpu.SemaphoreType.DMA((C,)),
            pltpu.SemaphoreType.DMA((C,)),
            pltpu.SemaphoreType.DMA((C,)),
            pltpu.SemaphoreType.DMA((C,)),
        ],
        compiler_params=pltpu.CompilerParams(collective_id=0),
    )(x)
